# baseline (speedup 1.0000x reference)
import functools

import jax
import jax.numpy as jnp
from jax.experimental import pallas as pl
from jax.experimental.pallas import tpu as pltpu

N = 2048
B = 8
LATENT = 4
NGMM = 10
NCHUNK = 16
CHUNK = N // NCHUNK

_PSHAPES = [
    (B, 1, jnp.int32),
    (B, 1, jnp.int32),
    (512, 128, jnp.float32),
    (1, 128, jnp.float32),
    (128, 32, jnp.float32),
    (1, 32, jnp.float32),
    (32, LATENT, jnp.float32),
    (1, LATENT, jnp.float32),
    (LATENT, 32, jnp.float32),
    (1, 32, jnp.float32),
    (32, NGMM, jnp.float32),
    (1, NGMM, jnp.float32),
]
_NP = len(_PSHAPES)


def _fused_body(*refs):
    (g_h, starts_h, W1_h, b1_h, W2_h, b2_h, W3_h, b3_h,
     We1_h, be1_h, We2_h, be2_h, x_hbm, adj_hbm,
     out_ref, gamma_ref) = refs[:16]
    pv = refs[16:16 + _NP]
    x_vmem, adj_vmem, h1_vmem, psems, xsem, sems = refs[16 + _NP:]
    f32 = jnp.float32

    small_hbm = (g_h, starts_h, W1_h, b1_h, W2_h, b2_h, W3_h, b3_h,
                 We1_h, be1_h, We2_h, be2_h)
    for i, src in enumerate(small_hbm):
        pltpu.make_async_copy(src, pv[i], psems.at[i]).start()
    pltpu.make_async_copy(x_hbm, x_vmem, xsem).start()
    for c in range(NCHUNK):
        pltpu.make_async_copy(
            adj_hbm.at[pl.ds(c * CHUNK, CHUNK), :],
            adj_vmem.at[pl.ds(c * CHUNK, CHUNK), :],
            sems.at[c],
        ).start()
    for i, src in enumerate(small_hbm):
        pltpu.make_async_copy(src, pv[i], psems.at[i]).wait()
    pltpu.make_async_copy(x_hbm, x_vmem, xsem).wait()

    (g_v, starts_v, W1_v, b1_v, W2_v, b2_v, W3_v, b3_v,
     We1_v, be1_v, We2_v, be2_v) = pv

    p1 = jnp.dot(x_vmem[...], W1_v[...], preferred_element_type=f32)
    b1 = b1_v[...]

    for c in range(NCHUNK):
        pltpu.make_async_copy(
            adj_hbm.at[pl.ds(c * CHUNK, CHUNK), :],
            adj_vmem.at[pl.ds(c * CHUNK, CHUNK), :],
            sems.at[c],
        ).wait()
        blk = adj_vmem[pl.ds(c * CHUNK, CHUNK), :]
        h1_vmem[pl.ds(c * CHUNK, CHUNK), :] = jnp.maximum(
            jnp.dot(blk, p1, preferred_element_type=f32) + b1, 0.0)

    adj = adj_vmem[...]
    p2 = jnp.dot(h1_vmem[...], W2_v[...], preferred_element_type=f32)
    h2 = jnp.maximum(jnp.dot(adj, p2, preferred_element_type=f32) + b2_v[...], 0.0)
    p3 = jnp.dot(h2, W3_v[...], preferred_element_type=f32)

    g = g_v[...]
    starts = starts_v[...]
    pos = jax.lax.broadcasted_iota(jnp.int32, (B, N), 1)
    mask = ((pos >= starts) & (pos < g)).astype(f32)
    q = jnp.dot(mask, adj, preferred_element_type=f32)
    sums = jnp.dot(q, p3, preferred_element_type=f32)
    counts = (g - starts).astype(f32)
    pooled = sums / counts + b3_v[...]

    hidden = jnp.maximum(jnp.dot(pooled, We1_v[...], preferred_element_type=f32) + be1_v[...], 0.0)
    logits = jnp.dot(hidden, We2_v[...], preferred_element_type=f32) + be2_v[...]
    m = jnp.max(logits, axis=1, keepdims=True)
    e = jnp.exp(logits - m)
    gamma = e / jnp.sum(e, axis=1, keepdims=True)

    out_ref[...] = pooled
    gamma_ref[...] = gamma


@functools.partial(jax.jit, static_argnames=("interpret",))
def _run(x, adj, g2, starts2, W1, b1, W2, b2, W3, b3, We1, be1, We2, be2,
         interpret=False):
    out, gamma = pl.pallas_call(
        _fused_body,
        out_shape=(
            jax.ShapeDtypeStruct((B, LATENT), jnp.float32),
            jax.ShapeDtypeStruct((B, NGMM), jnp.float32),
        ),
        in_specs=[pl.BlockSpec(memory_space=pl.ANY)] * 14,
        scratch_shapes=(
            [pltpu.VMEM((r, c), d) for (r, c, d) in _PSHAPES]
            + [pltpu.VMEM((N, 512), jnp.float32),
               pltpu.VMEM((N, N), jnp.float32),
               pltpu.VMEM((N, 128), jnp.float32),
               pltpu.SemaphoreType.DMA((_NP,)),
               pltpu.SemaphoreType.DMA,
               pltpu.SemaphoreType.DMA((NCHUNK,))]
        ),
        compiler_params=pltpu.CompilerParams(
            vmem_limit_bytes=100 * 1024 * 1024,
        ),
        interpret=interpret,
    )(g2, starts2,
      W1, b1.reshape(1, -1), W2, b2.reshape(1, -1), W3, b3.reshape(1, -1),
      We1, be1.reshape(1, -1), We2, be2.reshape(1, -1),
      x, adj)
    return out, gamma


def kernel(x, adj, graph_to_last_batch, W1, b1, W2, b2, W3, b3,
           We1, be1, We2, be2):
    g = graph_to_last_batch.astype(jnp.int32)
    starts = jnp.concatenate([jnp.zeros((1,), jnp.int32), g[:-1]])
    out, gamma = _run(x, adj, g.reshape(B, 1), starts.reshape(B, 1),
                      W1, b1, W2, b2, W3, b3, We1, be1, We2, be2)
    return (x, out, gamma)


# speedup vs baseline: 1.1376x; 1.1376x over previous
import functools

import jax
import jax.numpy as jnp
from jax.experimental import pallas as pl
from jax.experimental.pallas import tpu as pltpu

N = 2048
B = 8
LATENT = 4
NGMM = 10
NCHUNK = 16
CHUNK = N // NCHUNK


def _fused_body(x_ref, adj_hbm, g_ref, starts_ref,
                W1_ref, b1_ref, W2_ref, b2_ref, W3_ref, b3_ref,
                We1_ref, be1_ref, We2_ref, be2_ref,
                out_ref, gamma_ref,
                adj_vmem, h1_vmem, sems):
    f32 = jnp.float32

    for c in range(NCHUNK):
        pltpu.make_async_copy(
            adj_hbm.at[pl.ds(c * CHUNK, CHUNK), :],
            adj_vmem.at[pl.ds(c * CHUNK, CHUNK), :],
            sems.at[c],
        ).start()

    p1 = jnp.dot(x_ref[...], W1_ref[...], preferred_element_type=f32)
    b1 = b1_ref[...]

    for c in range(NCHUNK):
        pltpu.make_async_copy(
            adj_hbm.at[pl.ds(c * CHUNK, CHUNK), :],
            adj_vmem.at[pl.ds(c * CHUNK, CHUNK), :],
            sems.at[c],
        ).wait()
        blk = adj_vmem[pl.ds(c * CHUNK, CHUNK), :]
        h1_vmem[pl.ds(c * CHUNK, CHUNK), :] = jnp.maximum(
            jnp.dot(blk, p1, preferred_element_type=f32) + b1, 0.0)

    adj = adj_vmem[...]
    h1 = h1_vmem[...]

    p2 = jnp.dot(h1, W2_ref[...], preferred_element_type=f32)
    h2 = jnp.maximum(jnp.dot(adj, p2, preferred_element_type=f32) + b2_ref[...], 0.0)
    p3 = jnp.dot(h2, W3_ref[...], preferred_element_type=f32)

    g = g_ref[...]
    starts = starts_ref[...]
    pos = jax.lax.broadcasted_iota(jnp.int32, (B, N), 1)
    mask = ((pos >= starts) & (pos < g)).astype(f32)
    q = jnp.dot(mask, adj, preferred_element_type=f32)
    sums = jnp.dot(q, p3, preferred_element_type=f32)
    counts = (g - starts).astype(f32)
    pooled = sums / counts + b3_ref[...]

    hidden = jnp.maximum(jnp.dot(pooled, We1_ref[...], preferred_element_type=f32) + be1_ref[...], 0.0)
    logits = jnp.dot(hidden, We2_ref[...], preferred_element_type=f32) + be2_ref[...]
    m = jnp.max(logits, axis=1, keepdims=True)
    e = jnp.exp(logits - m)
    gamma = e / jnp.sum(e, axis=1, keepdims=True)

    out_ref[...] = pooled
    gamma_ref[...] = gamma


@functools.partial(jax.jit, static_argnames=("interpret",))
def _run(x, adj, g2, starts2, W1, b1, W2, b2, W3, b3, We1, be1, We2, be2,
         interpret=False):
    in_specs = [
        pl.BlockSpec(memory_space=pltpu.MemorySpace.VMEM),
        pl.BlockSpec(memory_space=pl.ANY),
    ] + [pl.BlockSpec(memory_space=pltpu.MemorySpace.VMEM)] * 12
    out, gamma = pl.pallas_call(
        _fused_body,
        out_shape=(
            jax.ShapeDtypeStruct((B, LATENT), jnp.float32),
            jax.ShapeDtypeStruct((B, NGMM), jnp.float32),
        ),
        in_specs=in_specs,
        scratch_shapes=[
            pltpu.VMEM((N, N), jnp.float32),
            pltpu.VMEM((N, 128), jnp.float32),
            pltpu.SemaphoreType.DMA((NCHUNK,)),
        ],
        compiler_params=pltpu.CompilerParams(
            vmem_limit_bytes=100 * 1024 * 1024,
        ),
        interpret=interpret,
    )(x, adj, g2, starts2,
      W1, b1.reshape(1, -1), W2, b2.reshape(1, -1), W3, b3.reshape(1, -1),
      We1, be1.reshape(1, -1), We2, be2.reshape(1, -1))
    return out, gamma


def kernel(x, adj, graph_to_last_batch, W1, b1, W2, b2, W3, b3,
           We1, be1, We2, be2):
    g = graph_to_last_batch.astype(jnp.int32)
    starts = jnp.concatenate([jnp.zeros((1,), jnp.int32), g[:-1]])
    out, gamma = _run(x, adj, g.reshape(B, 1), starts.reshape(B, 1),
                      W1, b1, W2, b2, W3, b3, We1, be1, We2, be2)
    return (x, out, gamma)
